# baseline (speedup 1.0000x reference)
import functools

import jax
import jax.numpy as jnp
from jax import lax
from jax.experimental import pallas as pl
from jax.experimental.pallas import tpu as pltpu
from jax.experimental.pallas import tpu_sc as plsc

N = 10000
E = 320000
IN = 128
HID = 8
HEADS = 8
OUT = 16

NC = 2
NS = 16
C = 128
TILES = NC * NS
CHUNKS_PER_TILE = 80
E_PAD = TILES * CHUNKS_PER_TILE * C
FAST_C = 0
CH_F = 112
CH_S = 48
NP = 10112
ROWS_PER_TILE = NP // NS


def _interleave_perm_matrix(fw):
    nat = fw + 16
    cols = []
    if fw == 64:
        for k in range(2):
            for i in range(16):
                cols.extend([32 * k + i, 32 * k + 16 + i])
        for i in range(16):
            cols.extend([64 + i, nat + i])
    else:
        for i in range(16):
            cols.extend([i, 16 + i])
    p = (jnp.array(cols, jnp.int32)[None, :] ==
         jnp.arange(nat, dtype=jnp.int32)[:, None])
    return p.astype(jnp.float32)


def _tc_layer1(x, w1p, a1l, a1r, m2, p1):
    def body(x_ref, w_ref, al_ref, ar_ref, m2_ref, p_ref, src_ref, dst_ref):
        feat = jnp.dot(x_ref[...], w_ref[...], preferred_element_type=jnp.float32)
        eld = jnp.dot(feat * al_ref[...][None, :], m2_ref[...],
                      preferred_element_type=jnp.float32)
        erd = jnp.dot(feat * ar_ref[...][None, :], m2_ref[...],
                      preferred_element_type=jnp.float32)
        full = jnp.concatenate([feat, eld], axis=1)
        src_ref[...] = jnp.dot(full, p_ref[...],
                               preferred_element_type=jnp.float32).astype(jnp.bfloat16)
        dst_ref[...] = erd

    return pl.pallas_call(
        body,
        out_shape=(
            jax.ShapeDtypeStruct((N, 96), jnp.bfloat16),
            jax.ShapeDtypeStruct((N, 16), jnp.float32),
        ),
    )(x, w1p, a1l, a1r, m2, p1)


def _tc_layer2(acc1, w2p, b1p, al2, ar2, p1, p2):
    def body(acc_ref, w_ref, b_ref, al_ref, ar_ref, p1_ref, p_ref,
             src_ref, dst_ref):
        s_int = (acc_ref[0, :N].astype(jnp.float32)
                 + acc_ref[1, :N].astype(jnp.float32))
        summed = jnp.dot(s_int, p1_ref[...].T,
                         preferred_element_type=jnp.float32)
        num = summed[:, 0:64]
        den16 = summed[:, 64:80] + 1e-9
        den64 = jnp.concatenate([den16, den16, den16, den16], axis=1)
        h = num / den64 + b_ref[...][None, :]
        h = jnp.where(h > 0, h, jnp.exp(jnp.minimum(h, 0.0)) - 1.0)
        feat2 = jnp.dot(h, w_ref[...], preferred_element_type=jnp.float32)
        el2 = jnp.dot(feat2, al_ref[...].T, preferred_element_type=jnp.float32)
        er2 = jnp.dot(feat2, ar_ref[...].T, preferred_element_type=jnp.float32)
        full = jnp.concatenate(
            [feat2, jnp.broadcast_to(el2, (N, 16))], axis=1)
        src_ref[...] = jnp.dot(full, p_ref[...],
                               preferred_element_type=jnp.float32).astype(jnp.bfloat16)
        dst_ref[...] = jnp.broadcast_to(er2, (N, 16))

    return pl.pallas_call(
        body,
        out_shape=(
            jax.ShapeDtypeStruct((N, 32), jnp.bfloat16),
            jax.ShapeDtypeStruct((N, 16), jnp.float32),
        ),
    )(acc1, w2p, b1p, al2, ar2, p1, p2)


def _tc_finalize(acc2, p2):
    def body(acc_ref, p_ref, out_ref):
        s_int = (acc_ref[0, :N].astype(jnp.float32)
                 + acc_ref[1, :N].astype(jnp.float32))
        summed = jnp.dot(s_int, p_ref[...].T,
                         preferred_element_type=jnp.float32)
        out_ref[...] = summed[:, 0:16] / (summed[:, 16:32] + 1e-9)

    return pl.pallas_call(
        body, out_shape=jax.ShapeDtypeStruct((N, OUT), jnp.float32)
    )(acc2, p2)


def _sc_edge_pass(tsrc, tdst, sidx2d, didx2d, sw, swb):
    fw = sw - 16
    mesh = plsc.VectorSubcoreMesh(
        core_axis_name="c", subcore_axis_name="s", num_cores=NC, num_subcores=NS
    )

    @functools.partial(
        pl.kernel,
        out_type=jax.ShapeDtypeStruct((NC, NP, swb), jnp.bfloat16),
        mesh=mesh,
        scratch_types=[
            pltpu.VMEM_SHARED((NP, sw), jnp.float32),
            pltpu.VMEM((2, C, swb), jnp.bfloat16),
            pltpu.VMEM((2, C, 16), jnp.float32),
            pltpu.VMEM((C, sw), jnp.float32),
            pltpu.VMEM((CH_F, C), jnp.int32),
            pltpu.VMEM((CH_F, C), jnp.int32),
            pltpu.SemaphoreType.DMA((2,)),
            pltpu.SemaphoreType.DMA((2,)),
        ],
        compiler_params=pltpu.CompilerParams(
            use_tc_tiling_on_sc=False, needs_layout_passes=False),
    )
    def run(tsrc_h, tdst_h, sidx_h, didx_h, out_h,
            acc, srcbuf, dstbuf, msgbuf, sidx, didx, gsem, dsem):
        c = lax.axis_index("c")
        s = lax.axis_index("s")
        is_fast = c == FAST_C
        ch = jnp.where(is_fast, CH_F, CH_S)
        base = jnp.where(is_fast, s * CH_F, NS * CH_F + s * CH_S)

        @pl.when(is_fast)
        def _():
            pltpu.sync_copy(sidx_h.at[pl.ds(base, CH_F)], sidx)
            pltpu.sync_copy(didx_h.at[pl.ds(base, CH_F)], didx)

        @pl.when(jnp.logical_not(is_fast))
        def _():
            pltpu.sync_copy(sidx_h.at[pl.ds(base, CH_S)], sidx.at[pl.ds(0, CH_S)])
            pltpu.sync_copy(didx_h.at[pl.ds(base, CH_S)], didx.at[pl.ds(0, CH_S)])

        def issue(j, b):
            pltpu.async_copy(tsrc_h.at[sidx.at[j]], srcbuf.at[b], gsem.at[b])
            pltpu.async_copy(tdst_h.at[didx.at[j]], dstbuf.at[b], dsem.at[b])

        def drain(j, b):
            pltpu.make_async_copy(tsrc_h.at[sidx.at[j]], srcbuf.at[b], gsem.at[b]).wait()
            pltpu.make_async_copy(tdst_h.at[didx.at[j]], dstbuf.at[b], dsem.at[b]).wait()

        issue(0, 0)

        zero = jnp.zeros((16,), jnp.float32)

        def zrow(i, carry):
            for k in range(sw // 16):
                msgbuf[i, pl.ds(k * 16, 16)] = zero
            return carry

        lax.fori_loop(0, C, zrow, 0)
        for k, sz in enumerate((128, 128, 128, 128, 120)):
            pltpu.sync_copy(
                msgbuf.at[pl.ds(0, sz)],
                acc.at[pl.ds(s * ROWS_PER_TILE + k * 128, sz)],
            )
        plsc.subcore_barrier()

        def pair(jj, carry):
            for b in range(2):
                j = 2 * jj + b

                @pl.when(j + 1 < ch)
                def _():
                    issue(j + 1, 1 - b)

                drain(j, b)

                @plsc.parallel_loop(0, C, unroll=4)
                def edge(i):
                    if fw == 64:
                        elb = srcbuf[b, i, pl.ds(64, 32)]
                        el, _ = plsc.unpack(
                            elb, format=plsc.PackFormat.INTERLEAVED,
                            preferred_element_type=jnp.float32)
                        ev = el + dstbuf[b, i, pl.ds(0, 16)]
                        ev = jnp.maximum(ev, ev * 0.2)
                        ee = jnp.exp(ev)
                        for k in range(2):
                            fb = srcbuf[b, i, pl.ds(32 * k, 32)]
                            lo, hi = plsc.unpack(
                                fb, format=plsc.PackFormat.INTERLEAVED,
                                preferred_element_type=jnp.float32)
                            msgbuf[i, pl.ds(32 * k, 16)] = ee * lo
                            msgbuf[i, pl.ds(32 * k + 16, 16)] = ee * hi
                    else:
                        fb = srcbuf[b, i, pl.ds(0, 32)]
                        f0, el = plsc.unpack(
                            fb, format=plsc.PackFormat.INTERLEAVED,
                            preferred_element_type=jnp.float32)
                        ev = el + dstbuf[b, i, pl.ds(0, 16)]
                        ev = jnp.maximum(ev, ev * 0.2)
                        ee = jnp.exp(ev)
                        msgbuf[i, pl.ds(0, 16)] = ee * f0
                    msgbuf[i, pl.ds(fw, 16)] = ee
                pltpu.sync_copy(msgbuf, acc.at[didx.at[j]], add=True)
            return carry

        lax.fori_loop(0, ch // 2, pair, 0)
        plsc.subcore_barrier()

        zvec = jnp.zeros((16,), jnp.float32)
        for kk, sz in enumerate((128, 128, 128, 128, 120)):
            row0 = s * ROWS_PER_TILE + kk * 128
            pltpu.sync_copy(acc.at[pl.ds(row0, sz)], msgbuf.at[pl.ds(0, sz)])

            @plsc.parallel_loop(0, sz, unroll=4)
            def cvt(r):
                for k in range(swb // 32):
                    a = msgbuf[r, pl.ds(32 * k, 16)]
                    if 32 * k + 32 <= sw:
                        bb = msgbuf[r, pl.ds(32 * k + 16, 16)]
                    else:
                        bb = zvec
                    srcbuf[0, r, pl.ds(32 * k, 32)] = plsc.pack(
                        a, bb, format=plsc.PackFormat.INTERLEAVED)

            pltpu.sync_copy(
                srcbuf.at[0, pl.ds(0, sz)],
                out_h.at[c, pl.ds(row0, sz)],
            )

    return run(tsrc, tdst, sidx2d, didx2d)


def kernel(features, edge_index, W1, al1, ar1, b1, W2, al2, ar2, b2):
    f32 = jnp.float32
    w1p = W1.reshape(IN, HEADS, HID).transpose(0, 2, 1).reshape(IN, HEADS * HID)
    a1l = al1.transpose(1, 0).reshape(HEADS * HID)
    a1r = ar1.transpose(1, 0).reshape(HEADS * HID)
    w2p = W2.reshape(HEADS, HID, OUT).transpose(1, 0, 2).reshape(HEADS * HID, OUT)
    b1p = b1.reshape(HEADS, HID).transpose(1, 0).reshape(HEADS * HID)
    m2 = (
        (jnp.arange(64)[:, None] % 8) == (jnp.arange(16)[None, :] % 8)
    ).astype(f32)

    src = edge_index[0].astype(jnp.int32)
    dst = edge_index[1].astype(jnp.int32)
    pad = E_PAD - E
    sidx2d = jnp.concatenate([src, jnp.zeros((pad,), jnp.int32)]).reshape(-1, C)
    pad_dst = N + (jnp.arange(pad, dtype=jnp.int32) % (NP - N))
    didx2d = jnp.concatenate([dst, pad_dst]).reshape(-1, C)

    p1 = _interleave_perm_matrix(64)
    p2 = _interleave_perm_matrix(16)
    src1, dst1 = _tc_layer1(features, w1p, a1l, a1r, m2, p1)
    acc1 = _sc_edge_pass(src1, dst1, sidx2d, didx2d, 80, 96)
    src2, dst2 = _tc_layer2(acc1, w2p, b1p, al2, ar2, p1, p2)
    acc2 = _sc_edge_pass(src2, dst2, sidx2d, didx2d, 32, 32)
    return _tc_finalize(acc2, p2)


# speedup vs baseline: 1.0533x; 1.0533x over previous
import functools

import jax
import jax.numpy as jnp
from jax import lax
from jax.experimental import pallas as pl
from jax.experimental.pallas import tpu as pltpu
from jax.experimental.pallas import tpu_sc as plsc

N = 10000
E = 320000
IN = 128
HID = 8
HEADS = 8
OUT = 16

NC = 2
NS = 16
C = 128
TILES = NC * NS
CHUNKS_PER_TILE = 80
E_PAD = TILES * CHUNKS_PER_TILE * C
FAST_C = 0
NP = 10112
ROWS_PER_TILE = NP // NS


def _interleave_perm_matrix(fw):
    nat = fw + 16
    cols = []
    if fw == 64:
        for k in range(2):
            for i in range(16):
                cols.extend([32 * k + i, 32 * k + 16 + i])
        for i in range(16):
            cols.extend([64 + i, nat + i])
    else:
        for i in range(16):
            cols.extend([i, 16 + i])
    p = (jnp.array(cols, jnp.int32)[None, :] ==
         jnp.arange(nat, dtype=jnp.int32)[:, None])
    return p.astype(jnp.float32)


def _tc_layer1(x, w1p, a1l, a1r, m2, p1):
    def body(x_ref, w_ref, al_ref, ar_ref, m2_ref, p_ref, src_ref, dst_ref):
        feat = jnp.dot(x_ref[...], w_ref[...], preferred_element_type=jnp.float32)
        eld = jnp.dot(feat * al_ref[...][None, :], m2_ref[...],
                      preferred_element_type=jnp.float32)
        erd = jnp.dot(feat * ar_ref[...][None, :], m2_ref[...],
                      preferred_element_type=jnp.float32)
        full = jnp.concatenate([feat, eld], axis=1)
        src_ref[...] = jnp.dot(full, p_ref[...],
                               preferred_element_type=jnp.float32).astype(jnp.bfloat16)
        dst_ref[...] = erd

    return pl.pallas_call(
        body,
        out_shape=(
            jax.ShapeDtypeStruct((N, 96), jnp.bfloat16),
            jax.ShapeDtypeStruct((N, 16), jnp.float32),
        ),
    )(x, w1p, a1l, a1r, m2, p1)


def _tc_layer2(acc1, w2p, b1p, al2, ar2, p2):
    def body(acc_ref, w_ref, b_ref, al_ref, ar_ref, p_ref, src_ref, dst_ref):
        summed = acc_ref[0, :N] + acc_ref[1, :N]
        num = summed[:, 0:64]
        den16 = summed[:, 64:80] + 1e-9
        den64 = jnp.concatenate([den16, den16, den16, den16], axis=1)
        h = num / den64 + b_ref[...][None, :]
        h = jnp.where(h > 0, h, jnp.exp(jnp.minimum(h, 0.0)) - 1.0)
        feat2 = jnp.dot(h, w_ref[...], preferred_element_type=jnp.float32)
        el2 = jnp.dot(feat2, al_ref[...].T, preferred_element_type=jnp.float32)
        er2 = jnp.dot(feat2, ar_ref[...].T, preferred_element_type=jnp.float32)
        full = jnp.concatenate(
            [feat2, jnp.broadcast_to(el2, (N, 16))], axis=1)
        src_ref[...] = jnp.dot(full, p_ref[...],
                               preferred_element_type=jnp.float32).astype(jnp.bfloat16)
        dst_ref[...] = jnp.broadcast_to(er2, (N, 16))

    return pl.pallas_call(
        body,
        out_shape=(
            jax.ShapeDtypeStruct((N, 32), jnp.bfloat16),
            jax.ShapeDtypeStruct((N, 16), jnp.float32),
        ),
    )(acc1, w2p, b1p, al2, ar2, p2)


def _tc_finalize(acc2):
    def body(acc_ref, out_ref):
        summed = acc_ref[0, :N] + acc_ref[1, :N]
        out_ref[...] = summed[:, 0:16] / (summed[:, 16:32] + 1e-9)

    return pl.pallas_call(
        body, out_shape=jax.ShapeDtypeStruct((N, OUT), jnp.float32)
    )(acc2)


def _sc_edge_pass(tsrc, tdst, sidx2d, didx2d, sw, swb, chf):
    fw = sw - 16
    chs = TILES * CHUNKS_PER_TILE // NS - chf
    mesh = plsc.VectorSubcoreMesh(
        core_axis_name="c", subcore_axis_name="s", num_cores=NC, num_subcores=NS
    )

    @functools.partial(
        pl.kernel,
        out_type=jax.ShapeDtypeStruct((NC, NP, sw), jnp.float32),
        mesh=mesh,
        scratch_types=[
            pltpu.VMEM_SHARED((NP, sw), jnp.float32),
            pltpu.VMEM((2, C, swb), jnp.bfloat16),
            pltpu.VMEM((2, C, 16), jnp.float32),
            pltpu.VMEM((C, sw), jnp.float32),
            pltpu.VMEM((chf, C), jnp.int32),
            pltpu.VMEM((chf, C), jnp.int32),
            pltpu.SemaphoreType.DMA((2,)),
            pltpu.SemaphoreType.DMA((2,)),
        ],
        compiler_params=pltpu.CompilerParams(
            use_tc_tiling_on_sc=False, needs_layout_passes=False),
    )
    def run(tsrc_h, tdst_h, sidx_h, didx_h, out_h,
            acc, srcbuf, dstbuf, msgbuf, sidx, didx, gsem, dsem):
        c = lax.axis_index("c")
        s = lax.axis_index("s")
        is_fast = c == FAST_C
        ch = jnp.where(is_fast, chf, chs)
        base = jnp.where(is_fast, s * chf, NS * chf + s * chs)

        @pl.when(is_fast)
        def _():
            pltpu.sync_copy(sidx_h.at[pl.ds(base, chf)], sidx)
            pltpu.sync_copy(didx_h.at[pl.ds(base, chf)], didx)

        @pl.when(jnp.logical_not(is_fast))
        def _():
            pltpu.sync_copy(sidx_h.at[pl.ds(base, chs)], sidx.at[pl.ds(0, chs)])
            pltpu.sync_copy(didx_h.at[pl.ds(base, chs)], didx.at[pl.ds(0, chs)])

        def issue(j, b):
            pltpu.async_copy(tsrc_h.at[sidx.at[j]], srcbuf.at[b], gsem.at[b])
            pltpu.async_copy(tdst_h.at[didx.at[j]], dstbuf.at[b], dsem.at[b])

        def drain(j, b):
            pltpu.make_async_copy(tsrc_h.at[sidx.at[j]], srcbuf.at[b], gsem.at[b]).wait()
            pltpu.make_async_copy(tdst_h.at[didx.at[j]], dstbuf.at[b], dsem.at[b]).wait()

        issue(0, 0)

        zero = jnp.zeros((16,), jnp.float32)

        def zrow(i, carry):
            for k in range(sw // 16):
                msgbuf[i, pl.ds(k * 16, 16)] = zero
            return carry

        lax.fori_loop(0, C, zrow, 0)
        for k, sz in enumerate((128, 128, 128, 128, 120)):
            pltpu.sync_copy(
                msgbuf.at[pl.ds(0, sz)],
                acc.at[pl.ds(s * ROWS_PER_TILE + k * 128, sz)],
            )
        plsc.subcore_barrier()

        def pair(jj, carry):
            for b in range(2):
                j = 2 * jj + b

                @pl.when(j + 1 < ch)
                def _():
                    issue(j + 1, 1 - b)

                drain(j, b)

                @plsc.parallel_loop(0, C, unroll=4)
                def edge(i):
                    if fw == 64:
                        elb = srcbuf[b, i, pl.ds(64, 32)]
                        el, _ = plsc.unpack(
                            elb, format=plsc.PackFormat.INTERLEAVED,
                            preferred_element_type=jnp.float32)
                        ev = el + dstbuf[b, i, pl.ds(0, 16)]
                        ev = jnp.maximum(ev, ev * 0.2)
                        ee = jnp.exp(ev)
                        for k in range(2):
                            fb = srcbuf[b, i, pl.ds(32 * k, 32)]
                            lo, hi = plsc.unpack(
                                fb, format=plsc.PackFormat.INTERLEAVED,
                                preferred_element_type=jnp.float32)
                            msgbuf[i, pl.ds(32 * k, 16)] = ee * lo
                            msgbuf[i, pl.ds(32 * k + 16, 16)] = ee * hi
                    else:
                        fb = srcbuf[b, i, pl.ds(0, 32)]
                        f0, el = plsc.unpack(
                            fb, format=plsc.PackFormat.INTERLEAVED,
                            preferred_element_type=jnp.float32)
                        ev = el + dstbuf[b, i, pl.ds(0, 16)]
                        ev = jnp.maximum(ev, ev * 0.2)
                        ee = jnp.exp(ev)
                        msgbuf[i, pl.ds(0, 16)] = ee * f0
                    msgbuf[i, pl.ds(fw, 16)] = ee
                pltpu.sync_copy(msgbuf, acc.at[didx.at[j]], add=True)
            return carry

        lax.fori_loop(0, ch // 2, pair, 0)
        plsc.subcore_barrier()
        pltpu.sync_copy(
            acc.at[pl.ds(s * ROWS_PER_TILE, ROWS_PER_TILE)],
            out_h.at[c, pl.ds(s * ROWS_PER_TILE, ROWS_PER_TILE)],
        )

    return run(tsrc, tdst, sidx2d, didx2d)


def kernel(features, edge_index, W1, al1, ar1, b1, W2, al2, ar2, b2):
    f32 = jnp.float32
    w1p = W1.reshape(IN, HEADS, HID).transpose(0, 2, 1).reshape(IN, HEADS * HID)
    a1l = al1.transpose(1, 0).reshape(HEADS * HID)
    a1r = ar1.transpose(1, 0).reshape(HEADS * HID)
    w2p = W2.reshape(HEADS, HID, OUT).transpose(1, 0, 2).reshape(HEADS * HID, OUT)
    b1p = b1.reshape(HEADS, HID).transpose(1, 0).reshape(HEADS * HID)
    m2 = (
        (jnp.arange(64)[:, None] % 8) == (jnp.arange(16)[None, :] % 8)
    ).astype(f32)

    src = edge_index[0].astype(jnp.int32)
    dst = edge_index[1].astype(jnp.int32)
    pad = E_PAD - E
    sidx2d = jnp.concatenate([src, jnp.zeros((pad,), jnp.int32)]).reshape(-1, C)
    pad_dst = N + (jnp.arange(pad, dtype=jnp.int32) % (NP - N))
    didx2d = jnp.concatenate([dst, pad_dst]).reshape(-1, C)

    p1 = _interleave_perm_matrix(64)
    p2 = _interleave_perm_matrix(16)
    src1, dst1 = _tc_layer1(features, w1p, a1l, a1r, m2, p1)
    acc1 = _sc_edge_pass(src1, dst1, sidx2d, didx2d, 80, 96, 124)
    src2, dst2 = _tc_layer2(acc1, w2p, b1p, al2, ar2, p2)
    acc2 = _sc_edge_pass(src2, dst2, sidx2d, didx2d, 32, 32, 104)
    return _tc_finalize(acc2)
